# Initial kernel scaffold; baseline (speedup 1.0000x reference)
import jax
import jax.numpy as jnp
from jax.experimental import pallas as pl


def kernel(x, edge_index, W1, b1, W2, b2, Wp1, bp1, Wp2, bp2, Wc, bc):
    raise NotImplementedError("write your pallas kernel here")


# speedup vs baseline: 9.3310x; 9.3310x over previous
import functools

import jax
import jax.numpy as jnp
from jax import lax
from jax.experimental import pallas as pl
from jax.experimental.pallas import tpu as pltpu
from jax.experimental.pallas import tpu_sc as plsc

_N = 10000
_E = 320000
_D = 128
_H = 128
_OUT = 4

_NC = 2
_NS = 16
_NW = _NC * _NS
_CH = 128
_EPT = -(-_E // (_NW * _CH)) * _CH
_EPAD = _EPT * _NW
_NSTEP = _EPT // _CH
_ZPT = 632
_AROWS = _ZPT * _NS


def _sc_mesh():
    return plsc.VectorSubcoreMesh(core_axis_name="c", subcore_axis_name="s")


@functools.partial(
    pl.kernel,
    out_type=jax.ShapeDtypeStruct((_NC, _AROWS, _H), jnp.float32),
    mesh=_sc_mesh(),
    scratch_types=[
        pltpu.VMEM((_CH,), jnp.int32),
        pltpu.VMEM((_CH, _H), jnp.float32),
        pltpu.VMEM_SHARED((_AROWS, _H), jnp.float32),
    ],
)
def _sc_deg(dst_hbm, ones_hbm, zer_hbm, out_hbm, dst_v, ones_v, acc_sh):
    cid = lax.axis_index("c")
    sid = lax.axis_index("s")
    wid = cid * _NS + sid
    pltpu.sync_copy(zer_hbm.at[pl.ds(sid * _ZPT, _ZPT)],
                    acc_sh.at[pl.ds(sid * _ZPT, _ZPT)])
    pltpu.sync_copy(ones_hbm, ones_v)
    plsc.subcore_barrier()

    def step(k, c):
        base = wid * _EPT + k * _CH
        pltpu.sync_copy(dst_hbm.at[pl.ds(base, _CH)], dst_v)
        pltpu.sync_copy(ones_v, acc_sh.at[dst_v], add=True)
        return c

    lax.fori_loop(0, _NSTEP, step, 0)
    plsc.subcore_barrier()
    pltpu.sync_copy(acc_sh.at[pl.ds(sid * _ZPT, _ZPT)],
                    out_hbm.at[cid, pl.ds(sid * _ZPT, _ZPT)])


@functools.partial(
    pl.kernel,
    out_type=jax.ShapeDtypeStruct((_NC, _AROWS, _H), jnp.float32),
    mesh=_sc_mesh(),
    scratch_types=[
        pltpu.VMEM((_CH,), jnp.int32),
        pltpu.VMEM((_CH,), jnp.int32),
        pltpu.VMEM((_CH, _H), jnp.float32),
        pltpu.VMEM_SHARED((_AROWS, _H), jnp.float32),
        pltpu.SemaphoreType.DMA,
    ],
)
def _sc_agg(y_hbm, src_hbm, dst_hbm, zer_hbm, out_hbm,
            src_v, dst_v, rows_v, acc_sh, sem):
    cid = lax.axis_index("c")
    sid = lax.axis_index("s")
    wid = cid * _NS + sid
    pltpu.sync_copy(zer_hbm.at[pl.ds(sid * _ZPT, _ZPT)],
                    acc_sh.at[pl.ds(sid * _ZPT, _ZPT)])
    plsc.subcore_barrier()

    def step(k, c):
        base = wid * _EPT + k * _CH
        pltpu.sync_copy(src_hbm.at[pl.ds(base, _CH)], src_v)
        pltpu.sync_copy(dst_hbm.at[pl.ds(base, _CH)], dst_v)
        pltpu.async_copy(y_hbm.at[src_v], rows_v, sem).wait()
        pltpu.sync_copy(rows_v, acc_sh.at[dst_v], add=True)
        return c

    lax.fori_loop(0, _NSTEP, step, 0)
    plsc.subcore_barrier()
    pltpu.sync_copy(acc_sh.at[pl.ds(sid * _ZPT, _ZPT)],
                    out_hbm.at[cid, pl.ds(sid * _ZPT, _ZPT)])


_R = 2000
_G = _N // _R


def _mm1_body(x_ref, w_ref, dp_ref, y_ref, dv_ref):
    dinv = lax.rsqrt(dp_ref[0, :, 0:1] + dp_ref[1, :, 0:1] + 1.0)
    dv_ref[...] = dinv
    xw = jnp.dot(x_ref[...], w_ref[...], preferred_element_type=jnp.float32)
    y_ref[...] = xw * dinv


def _mid_body(p_ref, y_ref, dv_ref, b_ref, w_ref, o_ref):
    dinv = dv_ref[...]
    h = jnp.maximum((p_ref[0] + p_ref[1] + y_ref[...]) * dinv + b_ref[...], 0.0)
    o_ref[...] = jnp.dot(h, w_ref[...], preferred_element_type=jnp.float32) * dinv


def _fin_body(p_ref, y_ref, dv_ref, b2_ref, imp_ref, wp1_ref, bp1_ref,
              wp2_ref, bp2_ref, wch_ref, wcx_ref, bc_ref, o_ref):
    dinv = dv_ref[...]
    h2 = jnp.maximum((p_ref[0] + p_ref[1] + y_ref[...]) * dinv + b2_ref[...], 0.0)
    xi = jnp.maximum(imp_ref[...] * wp1_ref[...] + bp1_ref[...], 0.0)
    xi = jnp.dot(xi, wp2_ref[...], preferred_element_type=jnp.float32) + bp2_ref[...]
    logits = (jnp.dot(h2, wch_ref[...], preferred_element_type=jnp.float32)
              + jnp.dot(xi, wcx_ref[...], preferred_element_type=jnp.float32)
              + bc_ref[...])
    m = jnp.max(logits, axis=1, keepdims=True)
    s = jnp.sum(jnp.exp(logits - m), axis=1, keepdims=True)
    o_ref[...] = logits - m - jnp.log(s)


_mm1 = pl.pallas_call(
    _mm1_body,
    grid=(_G,),
    in_specs=[
        pl.BlockSpec((_R, _D), lambda i: (i, 0)),
        pl.BlockSpec((_D, _H), lambda i: (0, 0)),
        pl.BlockSpec((_NC, _R, _H), lambda i: (0, i, 0)),
    ],
    out_specs=(pl.BlockSpec((_R, _H), lambda i: (i, 0)),
               pl.BlockSpec((_R, 1), lambda i: (i, 0))),
    out_shape=(jax.ShapeDtypeStruct((_N, _H), jnp.float32),
               jax.ShapeDtypeStruct((_N, 1), jnp.float32)),
)

_mid = pl.pallas_call(
    _mid_body,
    grid=(_G,),
    in_specs=[
        pl.BlockSpec((_NC, _R, _H), lambda i: (0, i, 0)),
        pl.BlockSpec((_R, _H), lambda i: (i, 0)),
        pl.BlockSpec((_R, 1), lambda i: (i, 0)),
        pl.BlockSpec((1, _H), lambda i: (0, 0)),
        pl.BlockSpec((_H, _H), lambda i: (0, 0)),
    ],
    out_specs=pl.BlockSpec((_R, _H), lambda i: (i, 0)),
    out_shape=jax.ShapeDtypeStruct((_N, _H), jnp.float32),
)

_fin = pl.pallas_call(
    _fin_body,
    grid=(_G,),
    in_specs=[
        pl.BlockSpec((_NC, _R, _H), lambda i: (0, i, 0)),
        pl.BlockSpec((_R, _H), lambda i: (i, 0)),
        pl.BlockSpec((_R, 1), lambda i: (i, 0)),
        pl.BlockSpec((1, _H), lambda i: (0, 0)),
        pl.BlockSpec((_R, 1), lambda i: (i, 0)),
        pl.BlockSpec((1, _H), lambda i: (0, 0)),
        pl.BlockSpec((1, _H), lambda i: (0, 0)),
        pl.BlockSpec((_H, _H), lambda i: (0, 0)),
        pl.BlockSpec((1, _H), lambda i: (0, 0)),
        pl.BlockSpec((_H, _OUT), lambda i: (0, 0)),
        pl.BlockSpec((_H, _OUT), lambda i: (0, 0)),
        pl.BlockSpec((1, _OUT), lambda i: (0, 0)),
    ],
    out_specs=pl.BlockSpec((_R, _OUT), lambda i: (i, 0)),
    out_shape=jax.ShapeDtypeStruct((_N, _OUT), jnp.float32),
)


def kernel(x, edge_index, W1, b1, W2, b2, Wp1, bp1, Wp2, bp2, Wc, bc):
    pad = _EPAD - _E
    src_p = jnp.concatenate([edge_index[0], jnp.zeros((pad,), jnp.int32)])
    dst_p = jnp.concatenate([edge_index[1], jnp.full((pad,), _N, jnp.int32)])
    ones128 = jnp.ones((_CH, _H), jnp.float32)
    zer128 = jnp.zeros((_AROWS, _H), jnp.float32)

    degp = _sc_deg(dst_p, ones128, zer128)
    y1, dinv = _mm1(x, W1, degp)
    p1 = _sc_agg(y1, src_p, dst_p, zer128)
    y2 = _mid(p1, y1, dinv, b1.reshape(1, -1), W2)
    p2 = _sc_agg(y2, src_p, dst_p, zer128)
    return _fin(p2, y2, dinv, b2.reshape(1, -1), x[:, _D - 1:_D],
                Wp1, bp1.reshape(1, -1), Wp2, bp2.reshape(1, -1),
                Wc[:_H], Wc[_H:], bc.reshape(1, -1))


# speedup vs baseline: 9.6195x; 1.0309x over previous
import functools

import jax
import jax.numpy as jnp
from jax import lax
from jax.experimental import pallas as pl
from jax.experimental.pallas import tpu as pltpu
from jax.experimental.pallas import tpu_sc as plsc

_N = 10000
_E = 320000
_D = 128
_H = 128
_OUT = 4

_NC = 2
_NS = 16
_NW = _NC * _NS
_CH = 128
_NSTEP = 80
_EPT = _NSTEP * _CH
_EPAD = _EPT * _NW
_ZPT = 632
_AROWS = _ZPT * _NS


def _sc_mesh():
    return plsc.VectorSubcoreMesh(core_axis_name="c", subcore_axis_name="s")


@functools.partial(
    pl.kernel,
    out_type=jax.ShapeDtypeStruct((_NC, _AROWS, _H), jnp.float32),
    mesh=_sc_mesh(),
    scratch_types=[
        pltpu.VMEM((_CH,), jnp.int32),
        pltpu.VMEM((_CH,), jnp.int32),
        pltpu.VMEM((_CH, _H), jnp.float32),
        pltpu.VMEM_SHARED((_AROWS, _H), jnp.float32),
        pltpu.SemaphoreType.DMA,
        pltpu.SemaphoreType.DMA,
    ],
)
def _sc_deg(dst_hbm, ones_hbm, zer_hbm, out_hbm,
            dst0, dst1, ones_v, acc_sh, sem0, sem1):
    cid = lax.axis_index("c")
    sid = lax.axis_index("s")
    wid = cid * _NS + sid
    base0 = wid * _EPT
    dsts = (dst0, dst1)
    sems = (sem0, sem1)
    pltpu.sync_copy(zer_hbm.at[pl.ds(sid * _ZPT, _ZPT)],
                    acc_sh.at[pl.ds(sid * _ZPT, _ZPT)])
    pltpu.sync_copy(ones_hbm, ones_v)
    for b in range(2):
        pltpu.async_copy(dst_hbm.at[pl.ds(base0 + b * _CH, _CH)],
                         dsts[b], sems[b])
    plsc.subcore_barrier()

    def pair(kk, c):
        for b in range(2):
            nxt = base0 + (kk * 2 + b + 2) * _CH
            pltpu.make_async_copy(dst_hbm.at[pl.ds(base0, _CH)],
                                  dsts[b], sems[b]).wait()
            pltpu.sync_copy(ones_v, acc_sh.at[dsts[b]], add=True)
            pltpu.async_copy(dst_hbm.at[pl.ds(nxt, _CH)], dsts[b], sems[b])
        return c

    lax.fori_loop(0, _NSTEP // 2 - 1, pair, 0)
    for b in range(2):
        pltpu.make_async_copy(dst_hbm.at[pl.ds(base0, _CH)],
                              dsts[b], sems[b]).wait()
        pltpu.sync_copy(ones_v, acc_sh.at[dsts[b]], add=True)
    plsc.subcore_barrier()
    pltpu.sync_copy(acc_sh.at[pl.ds(sid * _ZPT, _ZPT)],
                    out_hbm.at[cid, pl.ds(sid * _ZPT, _ZPT)])


@functools.partial(
    pl.kernel,
    out_type=jax.ShapeDtypeStruct((_NC, _AROWS, _H), jnp.float32),
    mesh=_sc_mesh(),
    scratch_types=[
        pltpu.VMEM((_CH,), jnp.int32),
        pltpu.VMEM((_CH,), jnp.int32),
        pltpu.VMEM((_CH,), jnp.int32),
        pltpu.VMEM((_CH,), jnp.int32),
        pltpu.VMEM((_CH, _H), jnp.float32),
        pltpu.VMEM((_CH, _H), jnp.float32),
        pltpu.VMEM_SHARED((_AROWS, _H), jnp.float32),
        pltpu.SemaphoreType.DMA,
        pltpu.SemaphoreType.DMA,
    ],
)
def _sc_agg(y_hbm, src_hbm, dst_hbm, zer_hbm, out_hbm,
            src0, src1, dst0, dst1, rows0, rows1, acc_sh, sem0, sem1):
    cid = lax.axis_index("c")
    sid = lax.axis_index("s")
    wid = cid * _NS + sid
    base0 = wid * _EPT
    srcs = (src0, src1)
    dsts = (dst0, dst1)
    rows = (rows0, rows1)
    sems = (sem0, sem1)
    pltpu.sync_copy(zer_hbm.at[pl.ds(sid * _ZPT, _ZPT)],
                    acc_sh.at[pl.ds(sid * _ZPT, _ZPT)])
    for b in range(2):
        pltpu.sync_copy(src_hbm.at[pl.ds(base0 + b * _CH, _CH)], srcs[b])
        pltpu.sync_copy(dst_hbm.at[pl.ds(base0 + b * _CH, _CH)], dsts[b])
        pltpu.async_copy(y_hbm.at[srcs[b]], rows[b], sems[b])
    plsc.subcore_barrier()

    def pair(kk, c):
        for b in range(2):
            nxt = base0 + (kk * 2 + b + 2) * _CH
            pltpu.make_async_copy(y_hbm.at[srcs[b]], rows[b], sems[b]).wait()
            pltpu.sync_copy(rows[b], acc_sh.at[dsts[b]], add=True)
            pltpu.sync_copy(src_hbm.at[pl.ds(nxt, _CH)], srcs[b])
            pltpu.sync_copy(dst_hbm.at[pl.ds(nxt, _CH)], dsts[b])
            pltpu.async_copy(y_hbm.at[srcs[b]], rows[b], sems[b])
        return c

    lax.fori_loop(0, _NSTEP // 2 - 1, pair, 0)
    for b in range(2):
        pltpu.make_async_copy(y_hbm.at[srcs[b]], rows[b], sems[b]).wait()
        pltpu.sync_copy(rows[b], acc_sh.at[dsts[b]], add=True)
    plsc.subcore_barrier()
    pltpu.sync_copy(acc_sh.at[pl.ds(sid * _ZPT, _ZPT)],
                    out_hbm.at[cid, pl.ds(sid * _ZPT, _ZPT)])


_R = 2000
_G = _N // _R


def _mm1_body(x_ref, w_ref, dp_ref, y_ref, dv_ref):
    dinv = lax.rsqrt(dp_ref[0, :, 0:1] + dp_ref[1, :, 0:1] + 1.0)
    dv_ref[...] = dinv
    xw = jnp.dot(x_ref[...], w_ref[...], preferred_element_type=jnp.float32)
    y_ref[...] = xw * dinv


def _mid_body(p_ref, y_ref, dv_ref, b_ref, w_ref, o_ref):
    dinv = dv_ref[...]
    h = jnp.maximum((p_ref[0] + p_ref[1] + y_ref[...]) * dinv + b_ref[...], 0.0)
    o_ref[...] = jnp.dot(h, w_ref[...], preferred_element_type=jnp.float32) * dinv


def _fin_body(p_ref, y_ref, dv_ref, b2_ref, imp_ref, wp1_ref, bp1_ref,
              wp2_ref, bp2_ref, wch_ref, wcx_ref, bc_ref, o_ref):
    dinv = dv_ref[...]
    h2 = jnp.maximum((p_ref[0] + p_ref[1] + y_ref[...]) * dinv + b2_ref[...], 0.0)
    xi = jnp.maximum(imp_ref[...] * wp1_ref[...] + bp1_ref[...], 0.0)
    xi = jnp.dot(xi, wp2_ref[...], preferred_element_type=jnp.float32) + bp2_ref[...]
    logits = (jnp.dot(h2, wch_ref[...], preferred_element_type=jnp.float32)
              + jnp.dot(xi, wcx_ref[...], preferred_element_type=jnp.float32)
              + bc_ref[...])
    m = jnp.max(logits, axis=1, keepdims=True)
    s = jnp.sum(jnp.exp(logits - m), axis=1, keepdims=True)
    o_ref[...] = logits - m - jnp.log(s)


_mm1 = pl.pallas_call(
    _mm1_body,
    grid=(_G,),
    in_specs=[
        pl.BlockSpec((_R, _D), lambda i: (i, 0)),
        pl.BlockSpec((_D, _H), lambda i: (0, 0)),
        pl.BlockSpec((_NC, _R, _H), lambda i: (0, i, 0)),
    ],
    out_specs=(pl.BlockSpec((_R, _H), lambda i: (i, 0)),
               pl.BlockSpec((_R, 1), lambda i: (i, 0))),
    out_shape=(jax.ShapeDtypeStruct((_N, _H), jnp.float32),
               jax.ShapeDtypeStruct((_N, 1), jnp.float32)),
)

_mid = pl.pallas_call(
    _mid_body,
    grid=(_G,),
    in_specs=[
        pl.BlockSpec((_NC, _R, _H), lambda i: (0, i, 0)),
        pl.BlockSpec((_R, _H), lambda i: (i, 0)),
        pl.BlockSpec((_R, 1), lambda i: (i, 0)),
        pl.BlockSpec((1, _H), lambda i: (0, 0)),
        pl.BlockSpec((_H, _H), lambda i: (0, 0)),
    ],
    out_specs=pl.BlockSpec((_R, _H), lambda i: (i, 0)),
    out_shape=jax.ShapeDtypeStruct((_N, _H), jnp.float32),
)

_fin = pl.pallas_call(
    _fin_body,
    grid=(_G,),
    in_specs=[
        pl.BlockSpec((_NC, _R, _H), lambda i: (0, i, 0)),
        pl.BlockSpec((_R, _H), lambda i: (i, 0)),
        pl.BlockSpec((_R, 1), lambda i: (i, 0)),
        pl.BlockSpec((1, _H), lambda i: (0, 0)),
        pl.BlockSpec((_R, 1), lambda i: (i, 0)),
        pl.BlockSpec((1, _H), lambda i: (0, 0)),
        pl.BlockSpec((1, _H), lambda i: (0, 0)),
        pl.BlockSpec((_H, _H), lambda i: (0, 0)),
        pl.BlockSpec((1, _H), lambda i: (0, 0)),
        pl.BlockSpec((_H, _OUT), lambda i: (0, 0)),
        pl.BlockSpec((_H, _OUT), lambda i: (0, 0)),
        pl.BlockSpec((1, _OUT), lambda i: (0, 0)),
    ],
    out_specs=pl.BlockSpec((_R, _OUT), lambda i: (i, 0)),
    out_shape=jax.ShapeDtypeStruct((_N, _OUT), jnp.float32),
)


def kernel(x, edge_index, W1, b1, W2, b2, Wp1, bp1, Wp2, bp2, Wc, bc):
    pad = _EPAD - _E
    src_p = jnp.concatenate([edge_index[0], jnp.zeros((pad,), jnp.int32)])
    dst_p = jnp.concatenate([edge_index[1], jnp.full((pad,), _N, jnp.int32)])
    ones128 = jnp.ones((_CH, _H), jnp.float32)
    zer128 = jnp.zeros((_AROWS, _H), jnp.float32)

    degp = _sc_deg(dst_p, ones128, zer128)
    y1, dinv = _mm1(x, W1, degp)
    p1 = _sc_agg(y1, src_p, dst_p, zer128)
    y2 = _mid(p1, y1, dinv, b1.reshape(1, -1), W2)
    p2 = _sc_agg(y2, src_p, dst_p, zer128)
    return _fin(p2, y2, dinv, b2.reshape(1, -1), x[:, _D - 1:_D],
                Wp1, bp1.reshape(1, -1), Wp2, bp2.reshape(1, -1),
                Wc[:_H], Wc[_H:], bc.reshape(1, -1))
